# baseline (speedup 1.0000x reference)
import jax
import jax.numpy as jnp
from jax import lax
from jax.experimental import pallas as pl
from jax.experimental.pallas import tpu as pltpu
from jax.experimental.pallas import tpu_sc as plsc

B, H, W, C = 8, 56, 56, 192
HW = H * W
NSPLIT = 2
BH = B // NSPLIT
ROWS = BH * HW
NW = 32
RPW = ROWS // NW
WPB = HW // RPW
CH = 56
NCH = RPW // CH
NBUF = 4
LANES = 16
IDXPAD = -(-RPW // LANES) * LANES


def _body(xf_hbm, r_hbm, out_hbm, idx_v, bufs, gsems, wsems):
    wid = lax.axis_index("s") * 2 + lax.axis_index("c")
    b = wid // WPB
    p0 = (wid % WPB) * RPW

    pltpu.sync_copy(r_hbm.at[pl.ds(p0, RPW)], idx_v.at[pl.ds(0, RPW)])
    off = b * HW
    for i in range(IDXPAD // LANES):
        sl = pl.ds(i * LANES, LANES)
        idx_v[sl] = idx_v[sl] + off

    base = wid * RPW

    def start_gather(c):
        return pltpu.async_copy(
            xf_hbm.at[idx_v.at[pl.ds(c * CH, CH)]], bufs[c % NBUF], gsems[c % NBUF]
        )

    def start_write(c):
        return pltpu.async_copy(
            bufs[c % NBUF], out_hbm.at[pl.ds(base + c * CH, CH)], wsems[c % NBUF]
        )

    gathers = [None] * NBUF
    writes = [None] * NBUF
    for j in range(min(NBUF, NCH)):
        gathers[j] = start_gather(j)
    for c in range(NCH):
        gathers[c % NBUF].wait()
        writes[c % NBUF] = start_write(c)
        n = c + NBUF
        if n < NCH:
            writes[n % NBUF].wait()
            gathers[n % NBUF] = start_gather(n)
    for j in range(max(0, NCH - NBUF), NCH):
        writes[j % NBUF].wait()


def _gather_half(xf, r):
    mesh = plsc.VectorSubcoreMesh(core_axis_name="c", subcore_axis_name="s")
    return pl.kernel(
        _body,
        out_type=jax.ShapeDtypeStruct((ROWS, C), jnp.float32),
        mesh=mesh,
        compiler_params=pltpu.CompilerParams(use_tc_tiling_on_sc=False),
        scratch_types=[
            pltpu.VMEM((IDXPAD,), jnp.int32),
            [pltpu.VMEM((CH, C), jnp.float32) for _ in range(NBUF)],
            [pltpu.SemaphoreType.DMA for _ in range(NBUF)],
            [pltpu.SemaphoreType.DMA for _ in range(NBUF)],
        ],
    )(xf, r)


@jax.jit
def kernel(x, r):
    r = r.astype(jnp.int32)
    xf = x.reshape(B * HW, C)
    outs = [
        _gather_half(lax.slice_in_dim(xf, h * ROWS, (h + 1) * ROWS, axis=0), r)
        for h in range(NSPLIT)
    ]
    out = lax.concatenate(outs, 0)
    return out.reshape(B, H, W, C)
